# Initial kernel scaffold; baseline (speedup 1.0000x reference)
import jax
import jax.numpy as jnp
from jax.experimental import pallas as pl


def kernel(x, edge_index, W1, b1, W2, b2, W3, b3):
    raise NotImplementedError("write your pallas kernel here")


# speedup vs baseline: 7.3713x; 7.3713x over previous
import functools

import jax
import jax.numpy as jnp
from jax import lax
from jax.experimental import pallas as pl
from jax.experimental.pallas import tpu as pltpu
from jax.experimental.pallas import tpu_sc as plsc

N = 10000
E = 320000
D = 128
L = 50

NC = 2
NS = 16
NW = NC * NS

NPAD = 10240
SLICE = NPAD // NS
EPC = E // NS

CHUNK = 128
EPT = E // NW
NCHUNK = 80
EPT_PAD = NCHUNK * CHUNK
ROWS_PT = NPAD // NS
ZROWS = 128
CPR = 8
CPC = NPAD // CPR

_mesh = plsc.VectorSubcoreMesh(core_axis_name="c", subcore_axis_name="s")
_sc_params = pltpu.CompilerParams(needs_layout_passes=False,
                                  use_tc_tiling_on_sc=False)


def _rsqrt_newton(d):
    nz = d > 0.0
    dd = jnp.maximum(d, 1.0)
    i = plsc.bitcast(dd, jnp.int32)
    i = jnp.int32(0x5F3759DF) - lax.shift_right_logical(i, 1)
    y = plsc.bitcast(i, jnp.float32)
    for _ in range(3):
        y = y * (1.5 - 0.5 * dd * y * y)
    return jnp.where(nz, y, 0.0)


@functools.partial(
    pl.kernel,
    compiler_params=_sc_params,
    out_type=[
        jax.ShapeDtypeStruct((NPAD,), jnp.float32),
        jax.ShapeDtypeStruct((NPAD,), jnp.float32),
    ],
    mesh=_mesh,
    scratch_types=[
        pltpu.VMEM((EPC,), jnp.int32),
        pltpu.VMEM((NPAD,), jnp.float32),
        pltpu.VMEM((NS, SLICE), jnp.float32),
        pltpu.VMEM_SHARED((NS, NPAD), jnp.float32),
    ],
)
def _norms_kernel(eflat_hbm, ns_hbm, nd_hbm, idx_v, hist_v, red_v, hist_sp):
    cid = lax.axis_index("c")
    sid = lax.axis_index("s")
    pltpu.sync_copy(eflat_hbm.at[pl.ds(cid * E + sid * EPC, EPC)], idx_v)

    zero16 = jnp.zeros((16,), jnp.float32)
    ones16 = jnp.ones((16,), jnp.float32)

    def zb(i, carry):
        hist_v[pl.ds(i * 16, 16)] = zero16
        return carry

    lax.fori_loop(0, NPAD // 16, zb, 0, unroll=4)

    def hb(i, carry):
        idx16 = idx_v[pl.ds(i * 16, 16)]
        plsc.addupdate_scatter(hist_v, [idx16], ones16)
        return carry

    lax.fori_loop(0, EPC // 16, hb, 0, unroll=4)

    pltpu.sync_copy(hist_v, hist_sp.at[sid])
    plsc.subcore_barrier()
    pltpu.sync_copy(hist_sp.at[:, pl.ds(sid * SLICE, SLICE)], red_v)

    def rb(k, carry):
        acc = red_v[0, pl.ds(k * 16, 16)]
        for s in range(1, NS):
            acc = acc + red_v[s, pl.ds(k * 16, 16)]
        hist_v[pl.ds(k * 16, 16)] = _rsqrt_newton(acc)
        return carry

    lax.fori_loop(0, SLICE // 16, rb, 0)

    @pl.when(cid == 0)
    def _():
        pltpu.sync_copy(hist_v.at[pl.ds(0, SLICE)],
                        ns_hbm.at[pl.ds(sid * SLICE, SLICE)])

    @pl.when(cid == 1)
    def _():
        pltpu.sync_copy(hist_v.at[pl.ds(0, SLICE)],
                        nd_hbm.at[pl.ds(sid * SLICE, SLICE)])


def _scale_body(x_ref, ns_ref, h_ref):
    h_ref[pl.ds(0, N), :] = x_ref[...] * ns_ref[...]
    h_ref[pl.ds(N, NPAD - N), :] = jnp.zeros((NPAD - N, D), jnp.float32)


def _scale_rows(x, nsrc_col):
    return pl.pallas_call(
        _scale_body,
        out_shape=jax.ShapeDtypeStruct((NPAD, D), jnp.float32),
    )(x, nsrc_col)


@functools.partial(
    pl.kernel,
    compiler_params=_sc_params,
    out_type=jax.ShapeDtypeStruct((NC, NPAD, D), jnp.float32),
    mesh=_mesh,
    scratch_types=[
        pltpu.VMEM((NCHUNK, CHUNK), jnp.int32),
        pltpu.VMEM((NCHUNK, CHUNK), jnp.int32),
        pltpu.VMEM((CHUNK, D), jnp.float32),
        pltpu.VMEM_SHARED((NPAD, D), jnp.float32),
        pltpu.SemaphoreType.DMA,
        pltpu.SemaphoreType.DMA,
    ],
)
def _edge_kernel(h_hbm, src_hbm, dst_hbm, agg_hbm,
                 srcv, dstv, rowsv, aggsp, sem_g, sem_s):
    cid = lax.axis_index("c")
    sid = lax.axis_index("s")
    wid = cid * NS + sid

    pltpu.sync_copy(src_hbm.at[wid], srcv)
    pltpu.sync_copy(dst_hbm.at[wid], dstv)

    zero16 = jnp.zeros((16,), jnp.float32)

    def zzb(i, carry):
        for k in range(D // 16):
            rowsv[i, pl.ds(k * 16, 16)] = zero16
        return carry

    lax.fori_loop(0, CHUNK, zzb, 0)

    for m in range(ROWS_PT // CHUNK):
        pltpu.sync_copy(rowsv, aggsp.at[pl.ds(sid * ROWS_PT + m * CHUNK, CHUNK)])
    plsc.subcore_barrier()

    def chunk_body(j, carry):
        pltpu.async_copy(h_hbm.at[srcv.at[j]], rowsv, sem_g).wait()
        pltpu.async_copy(rowsv, aggsp.at[dstv.at[j]], sem_s, add=True).wait()
        return carry

    lax.fori_loop(0, NCHUNK, chunk_body, 0)

    plsc.subcore_barrier()
    pltpu.sync_copy(aggsp.at[pl.ds(sid * ROWS_PT, ROWS_PT)],
                    agg_hbm.at[cid, pl.ds(sid * ROWS_PT, ROWS_PT)])


@functools.partial(
    pl.kernel,
    compiler_params=_sc_params,
    out_type=jax.ShapeDtypeStruct((NW * CPR, CPC), jnp.float32),
    mesh=_mesh,
    scratch_types=[
        pltpu.VMEM((NCHUNK, CHUNK), jnp.int32),
        pltpu.VMEM((NCHUNK, CHUNK), jnp.int32),
        pltpu.VMEM((NPAD,), jnp.float32),
        pltpu.VMEM((CPR, CPC), jnp.float32),
    ],
)
def _cpre_kernel(src_hbm, dst_hbm, nd_hbm, cpre_hbm, srcv, dstv, ndv, cprev):
    cid = lax.axis_index("c")
    sid = lax.axis_index("s")
    wid = cid * NS + sid

    pltpu.sync_copy(src_hbm.at[wid], srcv)
    pltpu.sync_copy(dst_hbm.at[wid], dstv)
    pltpu.sync_copy(nd_hbm, ndv)

    zero16 = jnp.zeros((16,), jnp.float32)

    def zcb(i, carry):
        r = lax.div(i, 10)
        c0 = lax.rem(i, 10) * 128
        for k in range(8):
            cprev[r, pl.ds(c0 + k * 16, 16)] = zero16
        return carry

    lax.fori_loop(0, CPR * 10, zcb, 0)

    def cb(j, carry):
        for k in range(CHUNK // 16):
            d16 = dstv[j, pl.ds(k * 16, 16)]
            s16 = srcv[j, pl.ds(k * 16, 16)]
            vals = plsc.load_gather(ndv, [d16])
            ri = lax.div(s16, CPC)
            ci = s16 - ri * CPC
            plsc.addupdate_scatter(cprev, [ri, ci], vals)
        return carry

    lax.fori_loop(0, NCHUNK, cb, 0)
    pltpu.sync_copy(cprev, cpre_hbm.at[pl.ds(wid * CPR, CPR)])


def _final_body(aggp_ref, cp_ref, nsr_ref, ndc_ref,
                w1_ref, b1_ref, w2_ref, b2_ref, w3_ref, b3_ref, out_ref):
    agg = (aggp_ref[0] + aggp_ref[1]) * ndc_ref[...]
    h1 = jnp.maximum(
        jnp.dot(agg, w1_ref[...], preferred_element_type=jnp.float32)
        + b1_ref[...], 0.0)
    c = jnp.sum(cp_ref[...], axis=0, keepdims=True) * nsr_ref[...]
    s = jnp.dot(c, h1, preferred_element_type=jnp.float32) * (1.0 / N)
    hg = jnp.dot(s, w2_ref[...], preferred_element_type=jnp.float32) + b2_ref[...]
    out_ref[...] = (
        jnp.dot(hg, w3_ref[...], preferred_element_type=jnp.float32)
        + b3_ref[...])


def _final(agg_part, cpre_part, nsrc_row, ndst_col, W1, b1, W2, b2, W3, b3):
    return pl.pallas_call(
        _final_body,
        out_shape=jax.ShapeDtypeStruct((1, L), jnp.float32),
    )(agg_part, cpre_part, nsrc_row, ndst_col,
      W1, b1.reshape(1, -1), W2, b2.reshape(1, -1), W3, b3.reshape(1, -1))


def kernel(x, edge_index, W1, b1, W2, b2, W3, b3):
    eflat = edge_index.reshape(2 * E)
    nsrc_pad, ndst_pad = _norms_kernel(eflat)
    h = _scale_rows(x, nsrc_pad[:N].reshape(N, 1))

    e2 = edge_index.reshape(2, NW, EPT)
    pad = jnp.full((2, NW, EPT_PAD - EPT), N, dtype=jnp.int32)
    ep = jnp.concatenate([e2, pad], axis=-1).reshape(2, NW, NCHUNK, CHUNK)

    agg_part = _edge_kernel(h, ep[0], ep[1])
    cpre_part = _cpre_kernel(ep[0], ep[1], ndst_pad).reshape(NW, NPAD)
    return _final(agg_part, cpre_part, nsrc_pad.reshape(1, NPAD),
                  ndst_pad.reshape(NPAD, 1), W1, b1, W2, b2, W3, b3)
